# Initial kernel scaffold; baseline (speedup 1.0000x reference)
import jax
import jax.numpy as jnp
from jax.experimental import pallas as pl


def kernel(charges, log_variance, batch_index, formal_charges):
    raise NotImplementedError("write your pallas kernel here")


# speedup vs baseline: 107.5032x; 107.5032x over previous
import functools

import jax
import jax.numpy as jnp
from jax import lax
from jax.experimental import pallas as pl
from jax.experimental.pallas import tpu as pltpu
from jax.experimental.pallas import tpu_sc as plsc

N = 1_600_000
G = 10_000
EPS = 1e-08

NC = 2
NS = 16
L = 16
NW = NC * NS
APW = N // NW
CS = 10_000
NCHUNK = APW // CS
GP = 10_240
GPW = GP // NW

_mesh = plsc.VectorSubcoreMesh(core_axis_name="c", subcore_axis_name="s")
_params = pltpu.CompilerParams(
    needs_layout_passes=False, use_tc_tiling_on_sc=False
)


def _wid():
    return lax.axis_index("s") * NC + lax.axis_index("c")


def _p1_body(ch_hbm, lv_hbm, bi_hbm, part_hbm, cbuf, lbuf, ibuf, accc, accv):
    wid = _wid()
    base = wid * APW

    def zero_body(i, _):
        s = pl.ds(i * L, L)
        accc[s] = jnp.zeros((L,), jnp.float32)
        accv[s] = jnp.zeros((L,), jnp.float32)
        return _

    lax.fori_loop(0, GP // L, zero_body, None)

    def chunk_body(k, _):
        off = base + k * CS
        pltpu.sync_copy(ch_hbm.at[pl.ds(off, CS)], cbuf)
        pltpu.sync_copy(lv_hbm.at[pl.ds(off, CS)], lbuf)
        pltpu.sync_copy(bi_hbm.at[pl.ds(off, CS)], ibuf)

        def inner(i, _):
            s = pl.ds(i * L, L)
            idx = ibuf[s]
            plsc.addupdate_scatter(accc, [idx], cbuf[s])
            plsc.addupdate_scatter(accv, [idx], jnp.exp(lbuf[s]))
            return _

        lax.fori_loop(0, CS // L, inner, None)
        return _

    lax.fori_loop(0, NCHUNK, chunk_body, None)
    pltpu.sync_copy(accc, part_hbm.at[2 * wid])
    pltpu.sync_copy(accv, part_hbm.at[2 * wid + 1])


_pass1 = functools.partial(
    pl.kernel,
    mesh=_mesh,
    compiler_params=_params,
    out_type=jax.ShapeDtypeStruct((2 * NW, GP), jnp.float32),
    scratch_types=[
        pltpu.VMEM((CS,), jnp.float32),
        pltpu.VMEM((CS,), jnp.float32),
        pltpu.VMEM((CS,), jnp.int32),
        pltpu.VMEM((GP,), jnp.float32),
        pltpu.VMEM((GP,), jnp.float32),
    ],
)(_p1_body)


def _p2_body(part_hbm, formal_hbm, scale_hbm, pbuf, fbuf, sbuf):
    wid = _wid()
    gbase = wid * GPW
    pltpu.sync_copy(part_hbm.at[:, pl.ds(gbase, GPW)], pbuf)
    pltpu.sync_copy(formal_hbm.at[pl.ds(gbase, GPW)], fbuf)

    def gbody(j, _):
        s = pl.ds(j * L, L)
        cs = jnp.zeros((L,), jnp.float32)
        vs = jnp.zeros((L,), jnp.float32)
        for t in range(NW):
            cs = cs + pbuf[2 * t, s]
            vs = vs + pbuf[2 * t + 1, s]
        sbuf[s] = (fbuf[s] - cs) / (vs + EPS)
        return _

    lax.fori_loop(0, GPW // L, gbody, None)
    pltpu.sync_copy(sbuf, scale_hbm.at[pl.ds(gbase, GPW)])


_pass2 = functools.partial(
    pl.kernel,
    mesh=_mesh,
    compiler_params=_params,
    out_type=jax.ShapeDtypeStruct((GP,), jnp.float32),
    scratch_types=[
        pltpu.VMEM((2 * NW, GPW), jnp.float32),
        pltpu.VMEM((GPW,), jnp.float32),
        pltpu.VMEM((GPW,), jnp.float32),
    ],
)(_p2_body)


def _p3_body(ch_hbm, lv_hbm, bi_hbm, scale_hbm, out_hbm, cbuf, lbuf, ibuf, sbuf):
    wid = _wid()
    base = wid * APW
    pltpu.sync_copy(scale_hbm, sbuf)

    def chunk_body(k, _):
        off = base + k * CS
        pltpu.sync_copy(ch_hbm.at[pl.ds(off, CS)], cbuf)
        pltpu.sync_copy(lv_hbm.at[pl.ds(off, CS)], lbuf)
        pltpu.sync_copy(bi_hbm.at[pl.ds(off, CS)], ibuf)

        def inner(i, _):
            s = pl.ds(i * L, L)
            w = plsc.load_gather(sbuf, [ibuf[s]])
            cbuf[s] = cbuf[s] + jnp.exp(lbuf[s]) * w
            return _

        lax.fori_loop(0, CS // L, inner, None)
        pltpu.sync_copy(cbuf, out_hbm.at[pl.ds(off, CS)])
        return _

    lax.fori_loop(0, NCHUNK, chunk_body, None)


_pass3 = functools.partial(
    pl.kernel,
    mesh=_mesh,
    compiler_params=_params,
    out_type=jax.ShapeDtypeStruct((N,), jnp.float32),
    scratch_types=[
        pltpu.VMEM((CS,), jnp.float32),
        pltpu.VMEM((CS,), jnp.float32),
        pltpu.VMEM((CS,), jnp.int32),
        pltpu.VMEM((GP,), jnp.float32),
    ],
)(_p3_body)


def kernel(charges, log_variance, batch_index, formal_charges):
    partials = _pass1(charges, log_variance, batch_index)
    formal_pad = jnp.pad(formal_charges.astype(jnp.float32), (0, GP - G))
    scale = _pass2(partials, formal_pad)
    return _pass3(charges, log_variance, batch_index, scale)


# speedup vs baseline: 195.8514x; 1.8218x over previous
import functools

import jax
import jax.numpy as jnp
from jax import lax
from jax.experimental import pallas as pl
from jax.experimental.pallas import tpu as pltpu
from jax.experimental.pallas import tpu_sc as plsc

N = 1_600_000
G = 10_000
EPS = 1e-08

NC = 2
NS = 16
L = 16
NW = NC * NS
APW = N // NW
CS = 10_000
NCHUNK = APW // CS
GP = 10_240
GPW = GP // NW

_mesh = plsc.VectorSubcoreMesh(core_axis_name="c", subcore_axis_name="s")
_params = pltpu.CompilerParams(
    needs_layout_passes=False, use_tc_tiling_on_sc=False
)


def _wid():
    return lax.axis_index("s") * NC + lax.axis_index("c")


def _p1_body(ch_hbm, lv_hbm, bi_hbm, part_hbm, cbuf, lbuf, ibuf, accc, accv):
    wid = _wid()
    base = wid * APW

    def zero_body(i, _):
        s = pl.ds(i * L, L)
        accc[s] = jnp.zeros((L,), jnp.float32)
        accv[s] = jnp.zeros((L,), jnp.float32)
        return _

    lax.fori_loop(0, GP // L, zero_body, None)

    lanes = lax.iota(jnp.int32, L) * (CS // L)

    def chunk_body(k, _):
        off = base + k * CS
        pltpu.sync_copy(ch_hbm.at[pl.ds(off, CS)], cbuf)
        pltpu.sync_copy(lv_hbm.at[pl.ds(off, CS)], lbuf)
        pltpu.sync_copy(bi_hbm.at[pl.ds(off, CS)], ibuf)

        def inner(i, _):
            pos = lanes + i
            idx = plsc.load_gather(ibuf, [pos])
            c = plsc.load_gather(cbuf, [pos])
            v = jnp.exp(plsc.load_gather(lbuf, [pos]))
            plsc.addupdate_scatter(accc, [idx], c)
            plsc.addupdate_scatter(accv, [idx], v)
            return _

        lax.fori_loop(0, CS // L, inner, None)
        return _

    lax.fori_loop(0, NCHUNK, chunk_body, None)
    pltpu.sync_copy(accc, part_hbm.at[2 * wid])
    pltpu.sync_copy(accv, part_hbm.at[2 * wid + 1])


_pass1 = functools.partial(
    pl.kernel,
    mesh=_mesh,
    compiler_params=_params,
    out_type=jax.ShapeDtypeStruct((2 * NW, GP), jnp.float32),
    scratch_types=[
        pltpu.VMEM((CS,), jnp.float32),
        pltpu.VMEM((CS,), jnp.float32),
        pltpu.VMEM((CS,), jnp.int32),
        pltpu.VMEM((GP,), jnp.float32),
        pltpu.VMEM((GP,), jnp.float32),
    ],
)(_p1_body)


def _p2_body(part_hbm, formal_hbm, scale_hbm, pbuf, fbuf, sbuf):
    wid = _wid()
    gbase = wid * GPW
    pltpu.sync_copy(part_hbm.at[:, pl.ds(gbase, GPW)], pbuf)
    pltpu.sync_copy(formal_hbm.at[pl.ds(gbase, GPW)], fbuf)

    def gbody(j, _):
        s = pl.ds(j * L, L)
        cs = jnp.zeros((L,), jnp.float32)
        vs = jnp.zeros((L,), jnp.float32)
        for t in range(NW):
            cs = cs + pbuf[2 * t, s]
            vs = vs + pbuf[2 * t + 1, s]
        sbuf[s] = (fbuf[s] - cs) / (vs + EPS)
        return _

    lax.fori_loop(0, GPW // L, gbody, None)
    pltpu.sync_copy(sbuf, scale_hbm.at[pl.ds(gbase, GPW)])


_pass2 = functools.partial(
    pl.kernel,
    mesh=_mesh,
    compiler_params=_params,
    out_type=jax.ShapeDtypeStruct((GP,), jnp.float32),
    scratch_types=[
        pltpu.VMEM((2 * NW, GPW), jnp.float32),
        pltpu.VMEM((GPW,), jnp.float32),
        pltpu.VMEM((GPW,), jnp.float32),
    ],
)(_p2_body)


def _p3_body(ch_hbm, lv_hbm, bi_hbm, scale_hbm, out_hbm, cbuf, lbuf, ibuf, sbuf):
    wid = _wid()
    base = wid * APW
    pltpu.sync_copy(scale_hbm, sbuf)

    def chunk_body(k, _):
        off = base + k * CS
        pltpu.sync_copy(ch_hbm.at[pl.ds(off, CS)], cbuf)
        pltpu.sync_copy(lv_hbm.at[pl.ds(off, CS)], lbuf)
        pltpu.sync_copy(bi_hbm.at[pl.ds(off, CS)], ibuf)

        def inner(i, _):
            s = pl.ds(i * L, L)
            w = plsc.load_gather(sbuf, [ibuf[s]])
            cbuf[s] = cbuf[s] + jnp.exp(lbuf[s]) * w
            return _

        lax.fori_loop(0, CS // L, inner, None)
        pltpu.sync_copy(cbuf, out_hbm.at[pl.ds(off, CS)])
        return _

    lax.fori_loop(0, NCHUNK, chunk_body, None)


_pass3 = functools.partial(
    pl.kernel,
    mesh=_mesh,
    compiler_params=_params,
    out_type=jax.ShapeDtypeStruct((N,), jnp.float32),
    scratch_types=[
        pltpu.VMEM((CS,), jnp.float32),
        pltpu.VMEM((CS,), jnp.float32),
        pltpu.VMEM((CS,), jnp.int32),
        pltpu.VMEM((GP,), jnp.float32),
    ],
)(_p3_body)


def kernel(charges, log_variance, batch_index, formal_charges):
    partials = _pass1(charges, log_variance, batch_index)
    formal_pad = jnp.pad(formal_charges.astype(jnp.float32), (0, GP - G))
    scale = _pass2(partials, formal_pad)
    return _pass3(charges, log_variance, batch_index, scale)


# speedup vs baseline: 199.9999x; 1.0212x over previous
import functools

import jax
import jax.numpy as jnp
from jax import lax
from jax.experimental import pallas as pl
from jax.experimental.pallas import tpu as pltpu
from jax.experimental.pallas import tpu_sc as plsc

N = 1_600_000
G = 10_000
EPS = 1e-08

NC = 2
NS = 16
L = 16
NW = NC * NS
APW = N // NW
CS = 10_000
NCHUNK = APW // CS
GP = 10_240
GPW = GP // NW
UNROLL = 5

_mesh = plsc.VectorSubcoreMesh(core_axis_name="c", subcore_axis_name="s")
_params = pltpu.CompilerParams(
    needs_layout_passes=False, use_tc_tiling_on_sc=False
)


def _wid():
    return lax.axis_index("s") * NC + lax.axis_index("c")


def _p1_body(ch_hbm, lv_hbm, bi_hbm, part_hbm, cbuf, lbuf, ibuf, accc, accv):
    wid = _wid()
    base = wid * APW

    def zero_body(i, _):
        s = pl.ds(i * L, L)
        accc[s] = jnp.zeros((L,), jnp.float32)
        accv[s] = jnp.zeros((L,), jnp.float32)
        return _

    lax.fori_loop(0, GP // L, zero_body, None)

    lanes = lax.iota(jnp.int32, L) * (CS // L)

    def chunk_body(k, _):
        off = base + k * CS
        pltpu.sync_copy(ch_hbm.at[pl.ds(off, CS)], cbuf)
        pltpu.sync_copy(lv_hbm.at[pl.ds(off, CS)], lbuf)
        pltpu.sync_copy(bi_hbm.at[pl.ds(off, CS)], ibuf)

        def inner(i, _):
            for u in range(UNROLL):
                pos = lanes + (i * UNROLL + u)
                idx = plsc.load_gather(ibuf, [pos])
                c = plsc.load_gather(cbuf, [pos])
                v = jnp.exp(plsc.load_gather(lbuf, [pos]))
                plsc.addupdate_scatter(accc, [idx], c)
                plsc.addupdate_scatter(accv, [idx], v)
            return _

        lax.fori_loop(0, CS // L // UNROLL, inner, None)
        return _

    lax.fori_loop(0, NCHUNK, chunk_body, None)
    pltpu.sync_copy(accc, part_hbm.at[2 * wid])
    pltpu.sync_copy(accv, part_hbm.at[2 * wid + 1])


_pass1 = functools.partial(
    pl.kernel,
    mesh=_mesh,
    compiler_params=_params,
    out_type=jax.ShapeDtypeStruct((2 * NW, GP), jnp.float32),
    scratch_types=[
        pltpu.VMEM((CS,), jnp.float32),
        pltpu.VMEM((CS,), jnp.float32),
        pltpu.VMEM((CS,), jnp.int32),
        pltpu.VMEM((GP,), jnp.float32),
        pltpu.VMEM((GP,), jnp.float32),
    ],
)(_p1_body)


def _p2_body(part_hbm, formal_hbm, scale_hbm, pbuf, fbuf, sbuf):
    wid = _wid()
    gbase = wid * GPW
    pltpu.sync_copy(part_hbm.at[:, pl.ds(gbase, GPW)], pbuf)
    pltpu.sync_copy(formal_hbm.at[pl.ds(gbase, GPW)], fbuf)

    def gbody(j, _):
        s = pl.ds(j * L, L)
        cs = jnp.zeros((L,), jnp.float32)
        vs = jnp.zeros((L,), jnp.float32)
        for t in range(NW):
            cs = cs + pbuf[2 * t, s]
            vs = vs + pbuf[2 * t + 1, s]
        sbuf[s] = (fbuf[s] - cs) / (vs + EPS)
        return _

    lax.fori_loop(0, GPW // L, gbody, None)
    pltpu.sync_copy(sbuf, scale_hbm.at[pl.ds(gbase, GPW)])


_pass2 = functools.partial(
    pl.kernel,
    mesh=_mesh,
    compiler_params=_params,
    out_type=jax.ShapeDtypeStruct((GP,), jnp.float32),
    scratch_types=[
        pltpu.VMEM((2 * NW, GPW), jnp.float32),
        pltpu.VMEM((GPW,), jnp.float32),
        pltpu.VMEM((GPW,), jnp.float32),
    ],
)(_p2_body)


def _p3_body(ch_hbm, lv_hbm, bi_hbm, scale_hbm, out_hbm, cbuf, lbuf, ibuf, sbuf):
    wid = _wid()
    base = wid * APW
    pltpu.sync_copy(scale_hbm, sbuf)

    def chunk_body(k, _):
        off = base + k * CS
        pltpu.sync_copy(ch_hbm.at[pl.ds(off, CS)], cbuf)
        pltpu.sync_copy(lv_hbm.at[pl.ds(off, CS)], lbuf)
        pltpu.sync_copy(bi_hbm.at[pl.ds(off, CS)], ibuf)

        def inner(i, _):
            for u in range(UNROLL):
                s = pl.ds((i * UNROLL + u) * L, L)
                w = plsc.load_gather(sbuf, [ibuf[s]])
                cbuf[s] = cbuf[s] + jnp.exp(lbuf[s]) * w
            return _

        lax.fori_loop(0, CS // L // UNROLL, inner, None)
        pltpu.sync_copy(cbuf, out_hbm.at[pl.ds(off, CS)])
        return _

    lax.fori_loop(0, NCHUNK, chunk_body, None)


_pass3 = functools.partial(
    pl.kernel,
    mesh=_mesh,
    compiler_params=_params,
    out_type=jax.ShapeDtypeStruct((N,), jnp.float32),
    scratch_types=[
        pltpu.VMEM((CS,), jnp.float32),
        pltpu.VMEM((CS,), jnp.float32),
        pltpu.VMEM((CS,), jnp.int32),
        pltpu.VMEM((GP,), jnp.float32),
    ],
)(_p3_body)


def kernel(charges, log_variance, batch_index, formal_charges):
    partials = _pass1(charges, log_variance, batch_index)
    formal_pad = jnp.pad(formal_charges.astype(jnp.float32), (0, GP - G))
    scale = _pass2(partials, formal_pad)
    return _pass3(charges, log_variance, batch_index, scale)


# speedup vs baseline: 217.7335x; 1.0887x over previous
import functools

import jax
import jax.numpy as jnp
from jax import lax
from jax.experimental import pallas as pl
from jax.experimental.pallas import tpu as pltpu
from jax.experimental.pallas import tpu_sc as plsc

N = 1_600_000
G = 10_000
EPS = 1e-08

NC = 2
NS = 16
L = 16
NW = NC * NS
APW = N // NW
CS = 10_000
NCHUNK = APW // CS
GP = 10_240
GPW = GP // NW
UNROLL = 5
EU = 25

_mesh = plsc.VectorSubcoreMesh(core_axis_name="c", subcore_axis_name="s")
_params = pltpu.CompilerParams(
    needs_layout_passes=False, use_tc_tiling_on_sc=False
)


def _wid():
    return lax.axis_index("s") * NC + lax.axis_index("c")


def _exp_loop(lbuf, vbuf):
    def body(i, _):
        for u in range(EU):
            s = pl.ds((i * EU + u) * L, L)
            vbuf[s] = jnp.exp(lbuf[s])
        return _

    lax.fori_loop(0, CS // L // EU, body, None)


def _p1_body(ch_hbm, lv_hbm, bi_hbm, part_hbm, cbuf, lbuf, ibuf, vbuf, accc, accv):
    wid = _wid()
    base = wid * APW

    def zero_body(i, _):
        s = pl.ds(i * L, L)
        accc[s] = jnp.zeros((L,), jnp.float32)
        accv[s] = jnp.zeros((L,), jnp.float32)
        return _

    lax.fori_loop(0, GP // L, zero_body, None)

    lanes = lax.iota(jnp.int32, L) * (CS // L)

    def chunk_body(k, _):
        off = base + k * CS
        pltpu.sync_copy(ch_hbm.at[pl.ds(off, CS)], cbuf)
        pltpu.sync_copy(lv_hbm.at[pl.ds(off, CS)], lbuf)
        pltpu.sync_copy(bi_hbm.at[pl.ds(off, CS)], ibuf)
        _exp_loop(lbuf, vbuf)

        def inner(i, _):
            for u in range(UNROLL):
                pos = lanes + (i * UNROLL + u)
                idx = plsc.load_gather(ibuf, [pos])
                c = plsc.load_gather(cbuf, [pos])
                v = plsc.load_gather(vbuf, [pos])
                plsc.addupdate_scatter(accc, [idx], c)
                plsc.addupdate_scatter(accv, [idx], v)
            return _

        lax.fori_loop(0, CS // L // UNROLL, inner, None)
        return _

    lax.fori_loop(0, NCHUNK, chunk_body, None)
    pltpu.sync_copy(accc, part_hbm.at[2 * wid])
    pltpu.sync_copy(accv, part_hbm.at[2 * wid + 1])


_pass1 = functools.partial(
    pl.kernel,
    mesh=_mesh,
    compiler_params=_params,
    out_type=jax.ShapeDtypeStruct((2 * NW, GP), jnp.float32),
    scratch_types=[
        pltpu.VMEM((CS,), jnp.float32),
        pltpu.VMEM((CS,), jnp.float32),
        pltpu.VMEM((CS,), jnp.int32),
        pltpu.VMEM((CS,), jnp.float32),
        pltpu.VMEM((GP,), jnp.float32),
        pltpu.VMEM((GP,), jnp.float32),
    ],
)(_p1_body)


def _p2_body(part_hbm, formal_hbm, scale_hbm, pbuf, fbuf, sbuf):
    wid = _wid()
    gbase = wid * GPW
    pltpu.sync_copy(part_hbm.at[:, pl.ds(gbase, GPW)], pbuf)
    pltpu.sync_copy(formal_hbm.at[pl.ds(gbase, GPW)], fbuf)

    def gbody(j, _):
        s = pl.ds(j * L, L)
        cs = jnp.zeros((L,), jnp.float32)
        vs = jnp.zeros((L,), jnp.float32)
        for t in range(NW):
            cs = cs + pbuf[2 * t, s]
            vs = vs + pbuf[2 * t + 1, s]
        sbuf[s] = (fbuf[s] - cs) / (vs + EPS)
        return _

    lax.fori_loop(0, GPW // L, gbody, None)
    pltpu.sync_copy(sbuf, scale_hbm.at[pl.ds(gbase, GPW)])


_pass2 = functools.partial(
    pl.kernel,
    mesh=_mesh,
    compiler_params=_params,
    out_type=jax.ShapeDtypeStruct((GP,), jnp.float32),
    scratch_types=[
        pltpu.VMEM((2 * NW, GPW), jnp.float32),
        pltpu.VMEM((GPW,), jnp.float32),
        pltpu.VMEM((GPW,), jnp.float32),
    ],
)(_p2_body)


def _p3_body(ch_hbm, lv_hbm, bi_hbm, scale_hbm, out_hbm, cbuf, lbuf, ibuf, vbuf, sbuf):
    wid = _wid()
    base = wid * APW
    pltpu.sync_copy(scale_hbm, sbuf)

    def chunk_body(k, _):
        off = base + k * CS
        pltpu.sync_copy(ch_hbm.at[pl.ds(off, CS)], cbuf)
        pltpu.sync_copy(lv_hbm.at[pl.ds(off, CS)], lbuf)
        pltpu.sync_copy(bi_hbm.at[pl.ds(off, CS)], ibuf)
        _exp_loop(lbuf, vbuf)

        def inner(i, _):
            for u in range(UNROLL):
                s = pl.ds((i * UNROLL + u) * L, L)
                w = plsc.load_gather(sbuf, [ibuf[s]])
                cbuf[s] = cbuf[s] + vbuf[s] * w
            return _

        lax.fori_loop(0, CS // L // UNROLL, inner, None)
        pltpu.sync_copy(cbuf, out_hbm.at[pl.ds(off, CS)])
        return _

    lax.fori_loop(0, NCHUNK, chunk_body, None)


_pass3 = functools.partial(
    pl.kernel,
    mesh=_mesh,
    compiler_params=_params,
    out_type=jax.ShapeDtypeStruct((N,), jnp.float32),
    scratch_types=[
        pltpu.VMEM((CS,), jnp.float32),
        pltpu.VMEM((CS,), jnp.float32),
        pltpu.VMEM((CS,), jnp.int32),
        pltpu.VMEM((CS,), jnp.float32),
        pltpu.VMEM((GP,), jnp.float32),
    ],
)(_p3_body)


def kernel(charges, log_variance, batch_index, formal_charges):
    partials = _pass1(charges, log_variance, batch_index)
    formal_pad = jnp.pad(formal_charges.astype(jnp.float32), (0, GP - G))
    scale = _pass2(partials, formal_pad)
    return _pass3(charges, log_variance, batch_index, scale)


# speedup vs baseline: 276.7673x; 1.2711x over previous
import functools

import jax
import jax.numpy as jnp
from jax import lax
from jax.experimental import pallas as pl
from jax.experimental.pallas import tpu as pltpu
from jax.experimental.pallas import tpu_sc as plsc

N = 1_600_000
G = 10_000
EPS = 1e-08

NC = 2
NS = 16
L = 16
NW = NC * NS
APW = N // NW
CS = 10_000
NCHUNK = APW // CS
GP = 10_240
GPW = GP // NW
UNROLL = 5
EU = 25

_mesh = plsc.VectorSubcoreMesh(core_axis_name="c", subcore_axis_name="s")
_params = pltpu.CompilerParams(
    needs_layout_passes=False, use_tc_tiling_on_sc=False
)


def _wid():
    return lax.axis_index("s") * NC + lax.axis_index("c")


def _exp_loop(lbuf, vbuf):
    @plsc.parallel_loop(0, CS // L, unroll=EU)
    def _(i):
        s = pl.ds(i * L, L)
        vbuf[s] = jnp.exp(lbuf[s])


def _p1_body(ch_hbm, lv_hbm, bi_hbm, part_hbm, cbuf, lbuf, ibuf, vbuf, accc, accv):
    wid = _wid()
    base = wid * APW

    @plsc.parallel_loop(0, GP // L, unroll=8)
    def _(i):
        s = pl.ds(i * L, L)
        accc[s] = jnp.zeros((L,), jnp.float32)
        accv[s] = jnp.zeros((L,), jnp.float32)

    lanes = lax.iota(jnp.int32, L) * (CS // L)

    def chunk_body(k, _):
        off = base + k * CS
        pltpu.sync_copy(ch_hbm.at[pl.ds(off, CS)], cbuf)
        pltpu.sync_copy(lv_hbm.at[pl.ds(off, CS)], lbuf)
        pltpu.sync_copy(bi_hbm.at[pl.ds(off, CS)], ibuf)
        _exp_loop(lbuf, vbuf)

        @plsc.parallel_loop(0, CS // L, unroll=UNROLL)
        def _(i):
            pos = lanes + i
            idx = plsc.load_gather(ibuf, [pos])
            c = plsc.load_gather(cbuf, [pos])
            v = plsc.load_gather(vbuf, [pos])
            plsc.addupdate_scatter(accc, [idx], c)
            plsc.addupdate_scatter(accv, [idx], v)

        return _

    lax.fori_loop(0, NCHUNK, chunk_body, None)
    pltpu.sync_copy(accc, part_hbm.at[2 * wid])
    pltpu.sync_copy(accv, part_hbm.at[2 * wid + 1])


_pass1 = functools.partial(
    pl.kernel,
    mesh=_mesh,
    compiler_params=_params,
    out_type=jax.ShapeDtypeStruct((2 * NW, GP), jnp.float32),
    scratch_types=[
        pltpu.VMEM((CS,), jnp.float32),
        pltpu.VMEM((CS,), jnp.float32),
        pltpu.VMEM((CS,), jnp.int32),
        pltpu.VMEM((CS,), jnp.float32),
        pltpu.VMEM((GP,), jnp.float32),
        pltpu.VMEM((GP,), jnp.float32),
    ],
)(_p1_body)


def _p2_body(part_hbm, formal_hbm, scale_hbm, pbuf, fbuf, sbuf):
    wid = _wid()
    gbase = wid * GPW
    pltpu.sync_copy(part_hbm.at[:, pl.ds(gbase, GPW)], pbuf)
    pltpu.sync_copy(formal_hbm.at[pl.ds(gbase, GPW)], fbuf)

    def gbody(j, _):
        s = pl.ds(j * L, L)
        cs = jnp.zeros((L,), jnp.float32)
        vs = jnp.zeros((L,), jnp.float32)
        for t in range(NW):
            cs = cs + pbuf[2 * t, s]
            vs = vs + pbuf[2 * t + 1, s]
        sbuf[s] = (fbuf[s] - cs) / (vs + EPS)
        return _

    lax.fori_loop(0, GPW // L, gbody, None)
    pltpu.sync_copy(sbuf, scale_hbm.at[pl.ds(gbase, GPW)])


_pass2 = functools.partial(
    pl.kernel,
    mesh=_mesh,
    compiler_params=_params,
    out_type=jax.ShapeDtypeStruct((GP,), jnp.float32),
    scratch_types=[
        pltpu.VMEM((2 * NW, GPW), jnp.float32),
        pltpu.VMEM((GPW,), jnp.float32),
        pltpu.VMEM((GPW,), jnp.float32),
    ],
)(_p2_body)


def _p3_body(ch_hbm, lv_hbm, bi_hbm, scale_hbm, out_hbm, cbuf, lbuf, ibuf, vbuf, sbuf):
    wid = _wid()
    base = wid * APW
    pltpu.sync_copy(scale_hbm, sbuf)

    def chunk_body(k, _):
        off = base + k * CS
        pltpu.sync_copy(ch_hbm.at[pl.ds(off, CS)], cbuf)
        pltpu.sync_copy(lv_hbm.at[pl.ds(off, CS)], lbuf)
        pltpu.sync_copy(bi_hbm.at[pl.ds(off, CS)], ibuf)
        _exp_loop(lbuf, vbuf)

        @plsc.parallel_loop(0, CS // L, unroll=UNROLL)
        def _(i):
            s = pl.ds(i * L, L)
            w = plsc.load_gather(sbuf, [ibuf[s]])
            cbuf[s] = cbuf[s] + vbuf[s] * w
        pltpu.sync_copy(cbuf, out_hbm.at[pl.ds(off, CS)])
        return _

    lax.fori_loop(0, NCHUNK, chunk_body, None)


_pass3 = functools.partial(
    pl.kernel,
    mesh=_mesh,
    compiler_params=_params,
    out_type=jax.ShapeDtypeStruct((N,), jnp.float32),
    scratch_types=[
        pltpu.VMEM((CS,), jnp.float32),
        pltpu.VMEM((CS,), jnp.float32),
        pltpu.VMEM((CS,), jnp.int32),
        pltpu.VMEM((CS,), jnp.float32),
        pltpu.VMEM((GP,), jnp.float32),
    ],
)(_p3_body)


def kernel(charges, log_variance, batch_index, formal_charges):
    partials = _pass1(charges, log_variance, batch_index)
    formal_pad = jnp.pad(formal_charges.astype(jnp.float32), (0, GP - G))
    scale = _pass2(partials, formal_pad)
    return _pass3(charges, log_variance, batch_index, scale)


# speedup vs baseline: 393.7190x; 1.4226x over previous
import functools

import jax
import jax.numpy as jnp
from jax import lax
from jax.experimental import pallas as pl
from jax.experimental.pallas import tpu as pltpu
from jax.experimental.pallas import tpu_sc as plsc

N = 1_600_000
G = 10_000
EPS = 1e-08

NC = 2
NS = 16
L = 16
NW = NC * NS
APW = N // NW
CS = 10_000
NCHUNK = APW // CS
GP = 10_240
GPW = GP // NW
UNROLL = 5
EU = 25

_mesh = plsc.VectorSubcoreMesh(core_axis_name="c", subcore_axis_name="s")
_params = pltpu.CompilerParams(
    needs_layout_passes=False, use_tc_tiling_on_sc=False
)


def _wid():
    return lax.axis_index("s") * NC + lax.axis_index("c")


def _exp_loop(lbuf, vbuf):
    @plsc.parallel_loop(0, CS // L, unroll=EU)
    def _(i):
        s = pl.ds(i * L, L)
        vbuf[s] = jnp.exp(lbuf[s])


def _p1_body(ch_hbm, lv_hbm, bi_hbm, part_hbm,
             cb0, lb0, ib0, cb1, lb1, ib1, vbuf, accc, accv, sm0, sm1):
    wid = _wid()
    base = wid * APW
    bufs = ((cb0, lb0, ib0, sm0), (cb1, lb1, ib1, sm1))

    def issue(k):
        cb, lb, ib, sem = bufs[k % 2]
        off = base + k * CS
        return (pltpu.async_copy(ch_hbm.at[pl.ds(off, CS)], cb, sem),
                pltpu.async_copy(lv_hbm.at[pl.ds(off, CS)], lb, sem),
                pltpu.async_copy(bi_hbm.at[pl.ds(off, CS)], ib, sem))

    pending = issue(0)

    @plsc.parallel_loop(0, GP // L, unroll=8)
    def _(i):
        s = pl.ds(i * L, L)
        accc[s] = jnp.zeros((L,), jnp.float32)
        accv[s] = jnp.zeros((L,), jnp.float32)

    lanes = lax.iota(jnp.int32, L) * (CS // L)

    for k in range(NCHUNK):
        for cp in pending:
            cp.wait()
        cb, lb, ib, _ = bufs[k % 2]
        if k + 1 < NCHUNK:
            pending = issue(k + 1)
        _exp_loop(lb, vbuf)

        @plsc.parallel_loop(0, CS // L, unroll=UNROLL)
        def _(i):
            pos = lanes + i
            idx = plsc.load_gather(ib, [pos])
            c = plsc.load_gather(cb, [pos])
            v = plsc.load_gather(vbuf, [pos])
            plsc.addupdate_scatter(accc, [idx], c)
            plsc.addupdate_scatter(accv, [idx], v)

    pltpu.sync_copy(accc, part_hbm.at[2 * wid])
    pltpu.sync_copy(accv, part_hbm.at[2 * wid + 1])


_pass1 = functools.partial(
    pl.kernel,
    mesh=_mesh,
    compiler_params=_params,
    out_type=jax.ShapeDtypeStruct((2 * NW, GP), jnp.float32),
    scratch_types=[
        pltpu.VMEM((CS,), jnp.float32),
        pltpu.VMEM((CS,), jnp.float32),
        pltpu.VMEM((CS,), jnp.int32),
        pltpu.VMEM((CS,), jnp.float32),
        pltpu.VMEM((CS,), jnp.float32),
        pltpu.VMEM((CS,), jnp.int32),
        pltpu.VMEM((CS,), jnp.float32),
        pltpu.VMEM((GP,), jnp.float32),
        pltpu.VMEM((GP,), jnp.float32),
        pltpu.SemaphoreType.DMA,
        pltpu.SemaphoreType.DMA,
    ],
)(_p1_body)


def _p2_body(part_hbm, formal_hbm, scale_hbm, pbuf, fbuf, sbuf):
    wid = _wid()
    gbase = wid * GPW
    pltpu.sync_copy(part_hbm.at[:, pl.ds(gbase, GPW)], pbuf)
    pltpu.sync_copy(formal_hbm.at[pl.ds(gbase, GPW)], fbuf)

    def gbody(j, _):
        s = pl.ds(j * L, L)
        cs = jnp.zeros((L,), jnp.float32)
        vs = jnp.zeros((L,), jnp.float32)
        for t in range(NW):
            cs = cs + pbuf[2 * t, s]
            vs = vs + pbuf[2 * t + 1, s]
        sbuf[s] = (fbuf[s] - cs) / (vs + EPS)
        return _

    lax.fori_loop(0, GPW // L, gbody, None)
    pltpu.sync_copy(sbuf, scale_hbm.at[pl.ds(gbase, GPW)])


_pass2 = functools.partial(
    pl.kernel,
    mesh=_mesh,
    compiler_params=_params,
    out_type=jax.ShapeDtypeStruct((GP,), jnp.float32),
    scratch_types=[
        pltpu.VMEM((2 * NW, GPW), jnp.float32),
        pltpu.VMEM((GPW,), jnp.float32),
        pltpu.VMEM((GPW,), jnp.float32),
    ],
)(_p2_body)


def _p3_body(ch_hbm, lv_hbm, bi_hbm, scale_hbm, out_hbm,
             cb0, lb0, ib0, cb1, lb1, ib1, vbuf, sbuf, sms, sm0, sm1):
    wid = _wid()
    base = wid * APW
    bufs = ((cb0, lb0, ib0, sm0), (cb1, lb1, ib1, sm1))

    def issue(k):
        cb, lb, ib, sem = bufs[k % 2]
        off = base + k * CS
        return (pltpu.async_copy(ch_hbm.at[pl.ds(off, CS)], cb, sem),
                pltpu.async_copy(lv_hbm.at[pl.ds(off, CS)], lb, sem),
                pltpu.async_copy(bi_hbm.at[pl.ds(off, CS)], ib, sem))

    scale_cp = pltpu.async_copy(scale_hbm, sbuf, sms)
    pending = issue(0)
    scale_cp.wait()
    writeback = [None, None]

    for k in range(NCHUNK):
        for cp in pending:
            cp.wait()
        cb, lb, ib, sem = bufs[k % 2]
        if k + 1 < NCHUNK:
            wb = writeback[(k + 1) % 2]
            if wb is not None:
                wb.wait()
            pending = issue(k + 1)
        _exp_loop(lb, vbuf)

        @plsc.parallel_loop(0, CS // L, unroll=UNROLL)
        def _(i):
            s = pl.ds(i * L, L)
            w = plsc.load_gather(sbuf, [ib[s]])
            cb[s] = cb[s] + vbuf[s] * w

        off = base + k * CS
        writeback[k % 2] = pltpu.async_copy(cb, out_hbm.at[pl.ds(off, CS)], sem)

    for wb in writeback:
        if wb is not None:
            wb.wait()


_pass3 = functools.partial(
    pl.kernel,
    mesh=_mesh,
    compiler_params=_params,
    out_type=jax.ShapeDtypeStruct((N,), jnp.float32),
    scratch_types=[
        pltpu.VMEM((CS,), jnp.float32),
        pltpu.VMEM((CS,), jnp.float32),
        pltpu.VMEM((CS,), jnp.int32),
        pltpu.VMEM((CS,), jnp.float32),
        pltpu.VMEM((CS,), jnp.float32),
        pltpu.VMEM((CS,), jnp.int32),
        pltpu.VMEM((CS,), jnp.float32),
        pltpu.VMEM((GP,), jnp.float32),
        pltpu.SemaphoreType.DMA,
        pltpu.SemaphoreType.DMA,
        pltpu.SemaphoreType.DMA,
    ],
)(_p3_body)


def kernel(charges, log_variance, batch_index, formal_charges):
    partials = _pass1(charges, log_variance, batch_index)
    formal_pad = jnp.pad(formal_charges.astype(jnp.float32), (0, GP - G))
    scale = _pass2(partials, formal_pad)
    return _pass3(charges, log_variance, batch_index, scale)
